# baseline (speedup 1.0000x reference)
import jax
import jax.numpy as jnp
from jax.experimental import pallas as pl
from jax.experimental.pallas import tpu as pltpu

NUM_CODES = 1024
NUM_CHANNELS = 64
BT = 2048


def _vq_logits_kernel(keys_ref, emb_ref, out_ref):
    k = keys_ref[...]
    e = emb_ref[...]
    k_sq = jnp.sum(k * k, axis=1, keepdims=True)
    e_sq = jnp.sum(e * e, axis=1, keepdims=True)
    a = jnp.concatenate([k + k, -k_sq, jnp.ones_like(k_sq)], axis=1)
    b = jnp.concatenate([e, jnp.ones_like(e_sq), -e_sq], axis=1)
    out_ref[...] = jax.lax.dot_general(
        a, b, (((1,), (1,)), ((), ())),
        preferred_element_type=jnp.float32,
    )


@jax.jit
def kernel(keys, embeddings):
    n_tokens = keys.shape[0]
    return pl.pallas_call(
        _vq_logits_kernel,
        grid=(n_tokens // BT,),
        in_specs=[
            pl.BlockSpec((BT, NUM_CHANNELS), lambda i: (i, 0)),
            pl.BlockSpec((NUM_CODES, NUM_CHANNELS), lambda i: (0, 0)),
        ],
        out_specs=pl.BlockSpec((BT, NUM_CODES), lambda i: (i, 0)),
        out_shape=jax.ShapeDtypeStruct((n_tokens, NUM_CODES), jnp.float32),
        compiler_params=pltpu.CompilerParams(
            dimension_semantics=("parallel",),
        ),
    )(keys, embeddings)


# speedup vs baseline: 1.3925x; 1.3925x over previous
import jax
import jax.numpy as jnp
from jax.experimental import pallas as pl
from jax.experimental.pallas import tpu as pltpu

NUM_CODES = 1024
NUM_CHANNELS = 64
BT = 2048


def _vq_logits_kernel(kt_ref, et_ref, out_ref):
    kt = kt_ref[...]
    et = et_ref[...]
    k_sq = jnp.sum(kt * kt, axis=0, keepdims=True)
    e_sq = jnp.sum(et * et, axis=0, keepdims=True)
    a_t = jnp.concatenate(
        [kt + kt, -k_sq, jnp.ones_like(k_sq)], axis=0
    )
    b_t = jnp.concatenate(
        [et, jnp.ones_like(e_sq), -e_sq], axis=0
    )
    out_ref[...] = jax.lax.dot_general(
        a_t, b_t, (((0,), (0,)), ((), ())),
        preferred_element_type=jnp.float32,
    )


@jax.jit
def kernel(keys, embeddings):
    n_tokens = keys.shape[0]
    kt = keys.T
    et = embeddings.T
    return pl.pallas_call(
        _vq_logits_kernel,
        grid=(n_tokens // BT,),
        in_specs=[
            pl.BlockSpec((NUM_CHANNELS, BT), lambda i: (0, i)),
            pl.BlockSpec((NUM_CHANNELS, NUM_CODES), lambda i: (0, 0)),
        ],
        out_specs=pl.BlockSpec((BT, NUM_CODES), lambda i: (i, 0)),
        out_shape=jax.ShapeDtypeStruct((n_tokens, NUM_CODES), jnp.float32),
        compiler_params=pltpu.CompilerParams(
            dimension_semantics=("arbitrary",),
        ),
    )(kt, et)
